# baseline (speedup 1.0000x reference)
import functools

import jax
import jax.numpy as jnp
from jax import lax
from jax.experimental import pallas as pl
from jax.experimental.pallas import tpu as pltpu
from jax.experimental.pallas import tpu_sc as plsc

N_TOPICS = 100000
N_CLASSES = 128
BATCH = 16384

NC, NS, L = 2, 16, 16
NW = NC * NS
BPW = BATCH // NW
CH = 256
NCHUNK = BPW // CH

_mesh = plsc.VectorSubcoreMesh(core_axis_name="c", subcore_axis_name="s")


@functools.partial(
    pl.kernel,
    out_type=jax.ShapeDtypeStruct((BATCH, N_CLASSES), jnp.float32),
    mesh=_mesh,
    scratch_types=[
        pltpu.VMEM((CH,), jnp.int32),
        pltpu.VMEM((CH,), jnp.float32),
        pltpu.VMEM((CH, N_CLASSES), jnp.float32),
        pltpu.VMEM((CH, N_CLASSES), jnp.float32),
        pltpu.SemaphoreType.DMA,
        pltpu.SemaphoreType.DMA,
        pltpu.SemaphoreType.DMA,
    ],
)
def _calibrate(logits_hbm, topics_hbm, loga_hbm, b_hbm, out_hbm,
               idx_v, loga_v, x_v, b_v, sem_b, sem_l, sem_x):
    wid = lax.axis_index("s") * NC + lax.axis_index("c")
    base = wid * BPW
    for ch in range(NCHUNK):
        off = base + ch * CH
        pltpu.sync_copy(topics_hbm.at[pl.ds(off, CH)], idx_v)
        cp_b = pltpu.async_copy(b_hbm.at[idx_v], b_v, sem_b)
        cp_l = pltpu.async_copy(loga_hbm.at[idx_v], loga_v, sem_l)
        cp_x = pltpu.async_copy(logits_hbm.at[pl.ds(off, CH)], x_v, sem_x)
        cp_l.wait()
        cp_x.wait()
        cp_b.wait()

        def grp_body(g, carry):
            sv = jnp.exp(loga_v[pl.ds(g * L, L)])
            for j in range(L):
                r = g * L + j
                s = jnp.full((L,), sv[j], jnp.float32)
                for c in range(N_CLASSES // L):
                    sl = (r, pl.ds(c * L, L))
                    b_v[sl] = x_v[sl] * s + b_v[sl]
            return carry

        lax.fori_loop(0, CH // L, grp_body, 0)
        pltpu.sync_copy(b_v, out_hbm.at[pl.ds(off, CH)])


def kernel(logits, topics, loga, b):
    return _calibrate(logits, topics.astype(jnp.int32), loga, b)


# speedup vs baseline: 1.2413x; 1.2413x over previous
import functools

import jax
import jax.numpy as jnp
from jax import lax
from jax.experimental import pallas as pl
from jax.experimental.pallas import tpu as pltpu
from jax.experimental.pallas import tpu_sc as plsc

N_TOPICS = 100000
N_CLASSES = 128
BATCH = 16384

NC, NS, L = 2, 16, 16
NW = NC * NS
BPW = BATCH // NW
CH = 128
NCH = BPW // CH
CREG = N_CLASSES // L

_mesh = plsc.VectorSubcoreMesh(core_axis_name="c", subcore_axis_name="s")


@functools.partial(
    pl.kernel,
    out_type=jax.ShapeDtypeStruct((BATCH, N_CLASSES), jnp.float32),
    mesh=_mesh,
    scratch_types=[
        pltpu.VMEM((BPW,), jnp.int32),
        pltpu.VMEM((BPW,), jnp.float32),
        pltpu.VMEM((2, CH, N_CLASSES), jnp.float32),
        pltpu.VMEM((2, CH, N_CLASSES), jnp.float32),
        pltpu.SemaphoreType.DMA,
        pltpu.SemaphoreType.DMA,
        pltpu.SemaphoreType.DMA,
        pltpu.SemaphoreType.DMA,
        pltpu.SemaphoreType.DMA,
        pltpu.SemaphoreType.DMA,
        pltpu.SemaphoreType.DMA,
    ],
)
def _calibrate(logits_hbm, topics_hbm, loga_hbm, b_hbm, out_hbm,
               idx_all, loga_all, x_v, b_v,
               sem_l, sem_x0, sem_x1, sem_b0, sem_b1, sem_o0, sem_o1):
    wid = lax.axis_index("s") * NC + lax.axis_index("c")
    base = wid * BPW
    sem_x = (sem_x0, sem_x1)
    sem_b = (sem_b0, sem_b1)
    sem_o = (sem_o0, sem_o1)

    pltpu.sync_copy(topics_hbm.at[pl.ds(base, BPW)], idx_all)
    cp_l = pltpu.async_copy(loga_hbm.at[idx_all], loga_all, sem_l)

    def issue(ch):
        k = ch % 2
        off = base + ch * CH
        cpx = pltpu.async_copy(logits_hbm.at[pl.ds(off, CH)], x_v.at[k], sem_x[k])
        cpb = pltpu.async_copy(b_hbm.at[idx_all.at[pl.ds(ch * CH, CH)]],
                               b_v.at[k], sem_b[k])
        return cpx, cpb

    pending = {0: issue(0)}
    outs = [None, None]
    cp_l.wait()
    for ch in range(NCH):
        k = ch % 2
        if ch + 1 < NCH:
            kn = (ch + 1) % 2
            if outs[kn] is not None:
                outs[kn].wait()
                outs[kn] = None
            pending[ch + 1] = issue(ch + 1)
        cpx, cpb = pending.pop(ch)
        cpx.wait()
        cpb.wait()

        def grp_body(g, carry):
            sv = jnp.exp(loga_all[pl.ds(ch * CH + g * L, L)])
            for j in range(L):
                r = g * L + j
                s = jnp.full((L,), sv[j], jnp.float32)
                for c in range(CREG):
                    sl = (k, r, pl.ds(c * L, L))
                    plsc.addupdate(b_v.at[sl], x_v[sl] * s)
            return carry

        lax.fori_loop(0, CH // L, grp_body, 0)
        outs[k] = pltpu.async_copy(b_v.at[k], out_hbm.at[pl.ds(base + ch * CH, CH)],
                                   sem_o[k])
    for cp in outs:
        if cp is not None:
            cp.wait()


def kernel(logits, topics, loga, b):
    return _calibrate(logits, topics.astype(jnp.int32), loga, b)
